# baseline (speedup 1.0000x reference)
import functools

import jax
import jax.numpy as jnp
from jax import lax
from jax.experimental import pallas as pl
from jax.experimental.pallas import tpu as pltpu
from jax.experimental.pallas import tpu_sc as plsc

B = 16384
D = 64
NBANDS = 7
OUT_D = NBANDS * D

NC = 2
NS = 16
NW = NC * NS
BPW = B // NW

_AS_BLOCK = 4096


def _assemble_body(p0_ref, p1_ref, p2_ref, xt_ref, w1_ref, b1_ref, w2_ref,
                   b2_ref, o_ref):
    o_ref[0 * 2 * D:1 * 2 * D, :] = p0_ref[...].T
    o_ref[1 * 2 * D:2 * 2 * D, :] = p1_ref[...].T
    o_ref[2 * 2 * D:3 * 2 * D, :] = p2_ref[...].T
    h = jnp.dot(w1_ref[...].T, xt_ref[...],
                preferred_element_type=jnp.float32)
    h = jnp.maximum(h + b1_ref[...], 0.0)
    o_ref[6 * D:, :] = (
        jnp.dot(w2_ref[...].T, h, preferred_element_type=jnp.float32)
        + b2_ref[...])


def _assemble_t(p0, p1, p2, xt, w1, b1, w2, b2):
    grid = (B // _AS_BLOCK,)
    band = pl.BlockSpec((_AS_BLOCK, 2 * D), lambda i: (i, 0))
    return pl.pallas_call(
        _assemble_body,
        grid=grid,
        in_specs=[
            band, band, band,
            pl.BlockSpec((D, _AS_BLOCK), lambda i: (0, i)),
            pl.BlockSpec((D, 32), lambda i: (0, 0)),
            pl.BlockSpec((32, 1), lambda i: (0, 0)),
            pl.BlockSpec((32, D), lambda i: (0, 0)),
            pl.BlockSpec((D, 1), lambda i: (0, 0)),
        ],
        out_specs=pl.BlockSpec((OUT_D, _AS_BLOCK), lambda i: (0, i)),
        out_shape=jax.ShapeDtypeStruct((OUT_D, B), jnp.float32),
        compiler_params=pltpu.CompilerParams(
            dimension_semantics=("parallel",)),
    )(p0, p1, p2, xt, w1, b1, w2, b2)


def _prep_chunk(v):
    return v if v <= 10000 else 26624


def _prep_body(xt_ref, o_ref):
    t = xt_ref[...].T
    half = t.shape[0] // 2
    o_ref[...] = jnp.concatenate([t[:half], t[half:]], axis=1)


def _prep(w):
    v = w.shape[0]
    chunk = _prep_chunk(v)
    nb = pl.cdiv(v, chunk)
    packed = pl.pallas_call(
        _prep_body,
        grid=(nb,),
        in_specs=[pl.BlockSpec((D, chunk), lambda i: (0, i))],
        out_specs=pl.BlockSpec((chunk // 2, 2 * D), lambda i: (i, 0)),
        out_shape=jax.ShapeDtypeStruct((nb * (chunk // 2), 2 * D),
                                       jnp.float32),
        compiler_params=pltpu.CompilerParams(
            dimension_semantics=("parallel",)),
    )(w.T)
    return packed.reshape(nb * chunk, D)


def _prep3_body(x0_ref, x1_ref, x2_ref, o0_ref, o1_ref, o2_ref):
    for x_ref, o_ref in ((x0_ref, o0_ref), (x1_ref, o1_ref),
                         (x2_ref, o2_ref)):
        t = x_ref[...].T
        half = t.shape[0] // 2
        o_ref[...] = jnp.concatenate([t[:half], t[half:]], axis=1)


def _prep3(w0, w1, w2):
    v = w0.shape[0]
    spec = pl.BlockSpec((D, v), lambda: (0, 0))
    ospec = pl.BlockSpec((v // 2, 2 * D), lambda: (0, 0))
    oshape = jax.ShapeDtypeStruct((v // 2, 2 * D), jnp.float32)
    outs = pl.pallas_call(
        _prep3_body,
        in_specs=[spec, spec, spec],
        out_specs=[ospec, ospec, ospec],
        out_shape=[oshape, oshape, oshape],
    )(w0.T, w1.T, w2.T)
    return [o.reshape(v, D) for o in outs]


def _permute_idx(i, v):
    ch = _prep_chunk(v)
    blk, j = i // ch, i % ch
    odd = j >= ch // 2
    jj = jnp.where(odd, j - ch // 2, j)
    return 2 * (blk * (ch // 2) + jj) + odd.astype(jnp.int32)


_sc_mesh = plsc.VectorSubcoreMesh(core_axis_name="c", subcore_axis_name="s")


def _make_sc_gather(nf):
    @functools.partial(
        pl.kernel,
        mesh=_sc_mesh,
        out_type=tuple(jax.ShapeDtypeStruct((B, 2 * D), jnp.float32)
                       for _ in range(nf // 2)),
        scratch_types=[
            pltpu.VMEM((BPW,), jnp.int32),
            pltpu.VMEM((BPW,), jnp.int32),
            pltpu.VMEM((BPW, D), jnp.float32),
            pltpu.VMEM((BPW, D), jnp.float32),
            pltpu.SemaphoreType.DMA,
            pltpu.SemaphoreType.DMA,
            pltpu.SemaphoreType.DMA,
        ],
        compiler_params=pltpu.CompilerParams(use_tc_tiling_on_sc=False),
    )
    def gather(*args):
        idxs = args[:nf]
        tabs = args[nf:2 * nf]
        outs = args[2 * nf:2 * nf + nf // 2]
        idx_a, idx_b, rows_a, rows_b, gsem_a, gsem_b, wsem = \
            args[2 * nf + nf // 2:]
        wid = lax.axis_index("s") * NC + lax.axis_index("c")
        base = wid * BPW
        idx_bufs = (idx_a, idx_b)
        row_bufs = (rows_a, rows_b)
        gsems = (gsem_a, gsem_b)
        gathers = []
        pltpu.sync_copy(idxs[0].at[pl.ds(base, BPW)], idx_a)
        gathers.append(pltpu.async_copy(tabs[0].at[idx_a], rows_a, gsem_a))
        writes = []
        for f in range(nf):
            if f < nf - 1:
                nidx = idx_bufs[(f + 1) % 2]
                pltpu.sync_copy(idxs[f + 1].at[pl.ds(base, BPW)], nidx)
            gathers[f].wait()
            if f >= 1:
                writes[f - 1].wait()
            if f < nf - 1:
                gathers.append(pltpu.async_copy(
                    tabs[f + 1].at[nidx], row_bufs[(f + 1) % 2],
                    gsems[(f + 1) % 2]))
            writes.append(pltpu.async_copy(
                row_bufs[f % 2],
                outs[f // 2].at[pl.ds(base, BPW), pl.ds((f % 2) * D, D)],
                wsem))
        writes[nf - 1].wait()

    return gather


_sc_gather4 = _make_sc_gather(4)
_sc_gather2 = _make_sc_gather(2)


def kernel(num_features, roast_level, test_method, price_tier, countries,
           countries_offsets, process, process_offsets, varietals,
           varietals_offsets, W_roast, W_test, W_price, W_countries,
           W_process, W_varietals, W1, b1, W2, b2):
    idx = [roast_level.astype(jnp.int32), test_method.astype(jnp.int32),
           price_tier.astype(jnp.int32), countries.astype(jnp.int32),
           process.astype(jnp.int32), varietals.astype(jnp.int32)]
    ws = (W_roast, W_test, W_price, W_countries, W_process, W_varietals)
    tabs = _prep3(W_roast, W_test, W_price)
    tabs += [_prep(W) for W in (W_countries, W_process, W_varietals)]
    perm = [_permute_idx(i, W.shape[0]) for i, W in zip(idx, ws)]
    p0, p1 = _sc_gather4(*perm[:4], *tabs[:4])
    (p2,) = _sc_gather2(*perm[4:], *tabs[4:])
    out_t = _assemble_t(p0, p1, p2, num_features.T, W1, b1.reshape(32, 1),
                        W2, b2.reshape(D, 1))
    return out_t.T


# speedup vs baseline: 1.0130x; 1.0130x over previous
import functools

import jax
import jax.numpy as jnp
from jax import lax
from jax.experimental import pallas as pl
from jax.experimental.pallas import tpu as pltpu
from jax.experimental.pallas import tpu_sc as plsc

B = 16384
D = 64
NBANDS = 7
OUT_D = NBANDS * D

NC = 2
NS = 16
NW = NC * NS
BPW = B // NW

_AS_BLOCK = 4096


def _assemble_body(p0_ref, p1_ref, p2_ref, xt_ref, w1_ref, b1_ref, w2_ref,
                   b2_ref, o_ref):
    o_ref[0 * 2 * D:1 * 2 * D, :] = p0_ref[...].T
    o_ref[1 * 2 * D:2 * 2 * D, :] = p1_ref[...].T
    o_ref[2 * 2 * D:3 * 2 * D, :] = p2_ref[...].T
    h = jnp.dot(w1_ref[...].T, xt_ref[...],
                preferred_element_type=jnp.float32)
    h = jnp.maximum(h + b1_ref[...], 0.0)
    o_ref[6 * D:, :] = (
        jnp.dot(w2_ref[...].T, h, preferred_element_type=jnp.float32)
        + b2_ref[...])


def _assemble_t(p0, p1, p2, xt, w1, b1, w2, b2):
    grid = (B // _AS_BLOCK,)
    band = pl.BlockSpec((_AS_BLOCK, 2 * D), lambda i: (i, 0))
    return pl.pallas_call(
        _assemble_body,
        grid=grid,
        in_specs=[
            band, band, band,
            pl.BlockSpec((D, _AS_BLOCK), lambda i: (0, i)),
            pl.BlockSpec((D, 32), lambda i: (0, 0)),
            pl.BlockSpec((32, 1), lambda i: (0, 0)),
            pl.BlockSpec((32, D), lambda i: (0, 0)),
            pl.BlockSpec((D, 1), lambda i: (0, 0)),
        ],
        out_specs=pl.BlockSpec((OUT_D, _AS_BLOCK), lambda i: (0, i)),
        out_shape=jax.ShapeDtypeStruct((OUT_D, B), jnp.float32),
        compiler_params=pltpu.CompilerParams(
            dimension_semantics=("parallel",)),
    )(p0, p1, p2, xt, w1, b1, w2, b2)


def _prep_chunk(v):
    return v if v <= 10000 else 12544


def _prep_body(xt_ref, o_ref):
    t = xt_ref[...].T
    half = t.shape[0] // 2
    o_ref[...] = jnp.concatenate([t[:half], t[half:]], axis=1)


def _prep(w):
    v = w.shape[0]
    chunk = _prep_chunk(v)
    nb = pl.cdiv(v, chunk)
    packed = pl.pallas_call(
        _prep_body,
        grid=(nb,),
        in_specs=[pl.BlockSpec((D, chunk), lambda i: (0, i))],
        out_specs=pl.BlockSpec((chunk // 2, 2 * D), lambda i: (i, 0)),
        out_shape=jax.ShapeDtypeStruct((nb * (chunk // 2), 2 * D),
                                       jnp.float32),
        compiler_params=pltpu.CompilerParams(
            dimension_semantics=("parallel",)),
    )(w.T)
    return packed.reshape(nb * chunk, D)


def _prep3_body(x0_ref, x1_ref, x2_ref, o0_ref, o1_ref, o2_ref):
    for x_ref, o_ref in ((x0_ref, o0_ref), (x1_ref, o1_ref),
                         (x2_ref, o2_ref)):
        t = x_ref[...].T
        half = t.shape[0] // 2
        o_ref[...] = jnp.concatenate([t[:half], t[half:]], axis=1)


def _prep3(w0, w1, w2):
    v = w0.shape[0]
    spec = pl.BlockSpec((D, v), lambda: (0, 0))
    ospec = pl.BlockSpec((v // 2, 2 * D), lambda: (0, 0))
    oshape = jax.ShapeDtypeStruct((v // 2, 2 * D), jnp.float32)
    outs = pl.pallas_call(
        _prep3_body,
        in_specs=[spec, spec, spec],
        out_specs=[ospec, ospec, ospec],
        out_shape=[oshape, oshape, oshape],
    )(w0.T, w1.T, w2.T)
    return [o.reshape(v, D) for o in outs]


def _permute_idx(i, v):
    ch = _prep_chunk(v)
    blk, j = i // ch, i % ch
    odd = j >= ch // 2
    jj = jnp.where(odd, j - ch // 2, j)
    return 2 * (blk * (ch // 2) + jj) + odd.astype(jnp.int32)


_sc_mesh = plsc.VectorSubcoreMesh(core_axis_name="c", subcore_axis_name="s")


def _make_sc_gather(nf):
    @functools.partial(
        pl.kernel,
        mesh=_sc_mesh,
        out_type=tuple(jax.ShapeDtypeStruct((B, 2 * D), jnp.float32)
                       for _ in range(nf // 2)),
        scratch_types=[
            pltpu.VMEM((BPW,), jnp.int32),
            pltpu.VMEM((BPW,), jnp.int32),
            pltpu.VMEM((BPW, D), jnp.float32),
            pltpu.VMEM((BPW, D), jnp.float32),
            pltpu.SemaphoreType.DMA,
            pltpu.SemaphoreType.DMA,
            pltpu.SemaphoreType.DMA,
        ],
        compiler_params=pltpu.CompilerParams(use_tc_tiling_on_sc=False),
    )
    def gather(*args):
        idxs = args[:nf]
        tabs = args[nf:2 * nf]
        outs = args[2 * nf:2 * nf + nf // 2]
        idx_a, idx_b, rows_a, rows_b, gsem_a, gsem_b, wsem = \
            args[2 * nf + nf // 2:]
        wid = lax.axis_index("s") * NC + lax.axis_index("c")
        base = wid * BPW
        idx_bufs = (idx_a, idx_b)
        row_bufs = (rows_a, rows_b)
        gsems = (gsem_a, gsem_b)
        gathers = []
        pltpu.sync_copy(idxs[0].at[pl.ds(base, BPW)], idx_a)
        gathers.append(pltpu.async_copy(tabs[0].at[idx_a], rows_a, gsem_a))
        writes = []
        for f in range(nf):
            if f < nf - 1:
                nidx = idx_bufs[(f + 1) % 2]
                pltpu.sync_copy(idxs[f + 1].at[pl.ds(base, BPW)], nidx)
            gathers[f].wait()
            if f >= 1:
                writes[f - 1].wait()
            if f < nf - 1:
                gathers.append(pltpu.async_copy(
                    tabs[f + 1].at[nidx], row_bufs[(f + 1) % 2],
                    gsems[(f + 1) % 2]))
            writes.append(pltpu.async_copy(
                row_bufs[f % 2],
                outs[f // 2].at[pl.ds(base, BPW), pl.ds((f % 2) * D, D)],
                wsem))
        writes[nf - 1].wait()

    return gather


_sc_gather4 = _make_sc_gather(4)
_sc_gather2 = _make_sc_gather(2)


def kernel(num_features, roast_level, test_method, price_tier, countries,
           countries_offsets, process, process_offsets, varietals,
           varietals_offsets, W_roast, W_test, W_price, W_countries,
           W_process, W_varietals, W1, b1, W2, b2):
    idx = [roast_level.astype(jnp.int32), test_method.astype(jnp.int32),
           price_tier.astype(jnp.int32), countries.astype(jnp.int32),
           process.astype(jnp.int32), varietals.astype(jnp.int32)]
    ws = (W_roast, W_test, W_price, W_countries, W_process, W_varietals)
    tabs = _prep3(W_roast, W_test, W_price)
    tabs += [_prep(W) for W in (W_countries, W_process, W_varietals)]
    perm = [_permute_idx(i, W.shape[0]) for i, W in zip(idx, ws)]
    p0, p1 = _sc_gather4(*perm[:4], *tabs[:4])
    (p2,) = _sc_gather2(*perm[4:], *tabs[4:])
    out_t = _assemble_t(p0, p1, p2, num_features.T, W1, b1.reshape(32, 1),
                        W2, b2.reshape(D, 1))
    return out_t.T
